# baseline (speedup 1.0000x reference)
import jax
import jax.numpy as jnp
from jax import lax
from jax.experimental import pallas as pl
from jax.experimental.pallas import tpu as pltpu
from jax.experimental.pallas import tpu_sc as plsc

_N = 10000
_E = 160000
_D = 256
_DH = 128
_NC = 2
_NS = 16
_RPT = _N // _NS

_ED_T = _E // (_NC * _NS)
_KD = 40
_NBD = _ED_T // _KD
_KS = 80
_NBS = 125
_ES_T = _KS * _NBS
_NACC = _N

_BN = 1000
_GRID = _N // _BN


def _mesh():
    return plsc.VectorSubcoreMesh(core_axis_name="c", subcore_axis_name="s")


_SC_PARAMS = pltpu.CompilerParams(use_tc_tiling_on_sc=False)


def _deg_body(dst_hbm, out_hbm, idx_v, ones_v, zb_v, acc_sh):
    c = lax.axis_index("c")
    s = lax.axis_index("s")
    wid = c * _NS + s

    def fill_ones(i, _):
        ones_v[i, :] = jnp.ones((16,), jnp.float32)
        return 0

    lax.fori_loop(0, _KD, fill_ones, 0)

    def fill_zb(i, _):
        zb_v[i, :] = jnp.zeros((16,), jnp.float32)
        return 0

    lax.fori_loop(0, _RPT, fill_zb, 0)

    pltpu.sync_copy(zb_v, acc_sh.at[pl.ds(s * _RPT, _RPT)])
    pltpu.sync_copy(dst_hbm.at[wid], idx_v)
    plsc.subcore_barrier()

    def body(b, _):
        pltpu.sync_copy(ones_v, acc_sh.at[idx_v.at[b]], add=True)
        return 0

    lax.fori_loop(0, _NBD, body, 0)
    plsc.subcore_barrier()

    pltpu.sync_copy(acc_sh.at[pl.ds(s * _RPT, _RPT)], zb_v)
    pltpu.sync_copy(zb_v, out_hbm.at[c, pl.ds(s * _RPT, _RPT)])


def _deg(dst_d):
    call = pl.kernel(
        _deg_body,
        out_type=jax.ShapeDtypeStruct((_NC, _N, 16), jnp.float32),
        mesh=_mesh(),
        scratch_types=[
            pltpu.VMEM((_NBD, _KD), jnp.int32),
            pltpu.VMEM((_KD, 16), jnp.float32),
            pltpu.VMEM((_RPT, 16), jnp.float32),
            pltpu.VMEM_SHARED((_N, 16), jnp.float32),
        ],
        compiler_params=_SC_PARAMS,
    )
    return call(dst_d)


def _scat_body(tbl_hbm, src_hbm, dst_hbm, out_hbm, isrc, idst, rbuf0, rbuf1, acc_sh, semg0, semg1, sems0, sems1):
    c = lax.axis_index("c")
    s = lax.axis_index("s")

    def fz(i, _):
        def fz2(j, _):
            rbuf0[i, pl.ds(j * 16, 16)] = jnp.zeros((16,), jnp.float32)
            return 0

        return lax.fori_loop(0, _DH // 16, fz2, 0)

    lax.fori_loop(0, _KS, fz, 0)
    nfull = _RPT // _KS
    tail = _RPT - nfull * _KS
    for k in range(nfull):
        pltpu.sync_copy(rbuf0, acc_sh.at[pl.ds(s * _RPT + k * _KS, _KS)])
    if tail:
        pltpu.sync_copy(
            rbuf0.at[pl.ds(0, tail)],
            acc_sh.at[pl.ds(s * _RPT + nfull * _KS, tail)],
        )

    pltpu.sync_copy(src_hbm.at[s], isrc)
    pltpu.sync_copy(dst_hbm.at[s], idst)

    coff = c * _N

    def off(i, _):
        def off2(j, _):
            isrc[i, pl.ds(j * 16, 16)] = isrc[i, pl.ds(j * 16, 16)] + coff
            return 0

        return lax.fori_loop(0, _KS // 16, off2, 0)

    lax.fori_loop(0, _NBS, off, 0)
    plsc.subcore_barrier()

    pltpu.async_copy(tbl_hbm.at[isrc.at[0]], rbuf0, semg0)

    def pair(i, _):
        b0 = 2 * i
        pltpu.make_async_copy(tbl_hbm.at[isrc.at[b0]], rbuf0, semg0).wait()
        pltpu.async_copy(tbl_hbm.at[isrc.at[b0 + 1]], rbuf1, semg1)
        pltpu.sync_copy(rbuf0, acc_sh.at[idst.at[b0]], add=True)
        pltpu.make_async_copy(tbl_hbm.at[isrc.at[b0 + 1]], rbuf1, semg1).wait()
        pltpu.async_copy(tbl_hbm.at[isrc.at[b0 + 2]], rbuf0, semg0)
        pltpu.sync_copy(rbuf1, acc_sh.at[idst.at[b0 + 1]], add=True)
        return 0

    lax.fori_loop(0, (_NBS - 1) // 2, pair, 0)
    pltpu.make_async_copy(tbl_hbm.at[isrc.at[_NBS - 1]], rbuf0, semg0).wait()
    pltpu.sync_copy(rbuf0, acc_sh.at[idst.at[_NBS - 1]], add=True)
    plsc.subcore_barrier()

    for k in range(nfull):
        rb = rbuf0 if k % 2 == 0 else rbuf1
        pltpu.sync_copy(acc_sh.at[pl.ds(s * _RPT + k * _KS, _KS)], rb)
        pltpu.sync_copy(rb, out_hbm.at[c, pl.ds(s * _RPT + k * _KS, _KS)])
    if tail:
        pltpu.sync_copy(
            acc_sh.at[pl.ds(s * _RPT + nfull * _KS, tail)],
            rbuf1.at[pl.ds(0, tail)],
        )
        pltpu.sync_copy(
            rbuf1.at[pl.ds(0, tail)],
            out_hbm.at[c, pl.ds(s * _RPT + nfull * _KS, tail)],
        )


def _scat(tbl, src_r, dst_r):
    call = pl.kernel(
        _scat_body,
        out_type=jax.ShapeDtypeStruct((_NC, _N, _DH), jnp.float32),
        mesh=_mesh(),
        scratch_types=[
            pltpu.VMEM((_NBS, _KS), jnp.int32),
            pltpu.VMEM((_NBS, _KS), jnp.int32),
            pltpu.VMEM((_KS, _DH), jnp.float32),
            pltpu.VMEM((_KS, _DH), jnp.float32),
            pltpu.VMEM_SHARED((_NACC, _DH), jnp.float32),
            pltpu.SemaphoreType.DMA,
            pltpu.SemaphoreType.DMA,
            pltpu.SemaphoreType.DMA,
            pltpu.SemaphoreType.DMA,
        ],
        compiler_params=_SC_PARAMS,
    )
    return call(tbl, src_r, dst_r)


def _dis_of(dp_ref):
    deg = 1.0 + dp_ref[0, :, 0] + dp_ref[1, :, 0]
    return lax.rsqrt(deg)[:, None]


def _k1_body(x_ref, w_ref, dp_ref, o_ref):
    h = jnp.dot(x_ref[...], w_ref[...], preferred_element_type=jnp.float32)
    hp = h * _dis_of(dp_ref)
    o_ref[0] = hp[:, :_DH]
    o_ref[1] = hp[:, _DH:]


def _k1(x, W1, dp):
    return pl.pallas_call(
        _k1_body,
        grid=(_GRID,),
        in_specs=[
            pl.BlockSpec((_BN, _D), lambda i: (i, 0)),
            pl.BlockSpec((_D, _D), lambda i: (0, 0)),
            pl.BlockSpec((_NC, _BN, 16), lambda i: (0, i, 0)),
        ],
        out_specs=pl.BlockSpec((_NC, _BN, _DH), lambda i: (0, i, 0)),
        out_shape=jax.ShapeDtypeStruct((_NC, _N, _DH), jnp.float32),
    )(x, W1, dp)


def _k2_body(s_ref, hp_ref, dp_ref, b_ref, w_ref, o_ref):
    dis = _dis_of(dp_ref)
    ssum = jnp.concatenate([s_ref[0] + hp_ref[0], s_ref[1] + hp_ref[1]], axis=1)
    z = jnp.maximum(ssum * dis + b_ref[...], 0.0)
    h = jnp.dot(z, w_ref[...], preferred_element_type=jnp.float32)
    hp = h * dis
    o_ref[0] = hp[:, :_DH]
    o_ref[1] = hp[:, _DH:]


def _k2(s1, h1p, dp, b1, W2):
    return pl.pallas_call(
        _k2_body,
        grid=(_GRID,),
        in_specs=[
            pl.BlockSpec((_NC, _BN, _DH), lambda i: (0, i, 0)),
            pl.BlockSpec((_NC, _BN, _DH), lambda i: (0, i, 0)),
            pl.BlockSpec((_NC, _BN, 16), lambda i: (0, i, 0)),
            pl.BlockSpec((1, _D), lambda i: (0, 0)),
            pl.BlockSpec((_D, _D), lambda i: (0, 0)),
        ],
        out_specs=pl.BlockSpec((_NC, _BN, _DH), lambda i: (0, i, 0)),
        out_shape=jax.ShapeDtypeStruct((_NC, _N, _DH), jnp.float32),
    )(s1, h1p, dp, b1, W2)


def _k3_body(s_ref, hp_ref, dp_ref, b_ref, o_ref):
    dis = _dis_of(dp_ref)
    ssum = jnp.concatenate([s_ref[0] + hp_ref[0], s_ref[1] + hp_ref[1]], axis=1)
    o_ref[...] = ssum * dis + b_ref[...]


def _k3(s2, h2p, dp, b2):
    return pl.pallas_call(
        _k3_body,
        grid=(_GRID,),
        in_specs=[
            pl.BlockSpec((_NC, _BN, _DH), lambda i: (0, i, 0)),
            pl.BlockSpec((_NC, _BN, _DH), lambda i: (0, i, 0)),
            pl.BlockSpec((_NC, _BN, 16), lambda i: (0, i, 0)),
            pl.BlockSpec((1, _D), lambda i: (0, 0)),
        ],
        out_specs=pl.BlockSpec((_BN, _D), lambda i: (i, 0)),
        out_shape=jax.ShapeDtypeStruct((_N, _D), jnp.float32),
    )(s2, h2p, dp, b2)


def kernel(x, edge_index, W1, b1, W2, b2):
    src = edge_index[0]
    dst = edge_index[1]
    src_r = src.reshape(_NS, _NBS, _KS)
    dst_r = dst.reshape(_NS, _NBS, _KS)
    dst_d = dst.reshape(_NC * _NS, _NBD, _KD)

    degp = _deg(dst_d)
    h1p = _k1(x, W1, degp)
    s1 = _scat(h1p.reshape(_NC * _N, _DH), src_r, dst_r)
    h2p = _k2(s1, h1p, degp, b1.reshape(1, _D), W2)
    s2 = _scat(h2p.reshape(_NC * _N, _DH), src_r, dst_r)
    return _k3(s2, h2p, degp, b2.reshape(1, _D))


# speedup vs baseline: 1.3879x; 1.3879x over previous
import jax
import jax.numpy as jnp
from jax import lax
from jax.experimental import pallas as pl
from jax.experimental.pallas import tpu as pltpu
from jax.experimental.pallas import tpu_sc as plsc

_N = 10000
_E = 160000
_D = 256
_DH = 128
_NC = 2
_NS = 16
_RPT = _N // _NS

_ED_T = _E // (_NC * _NS)
_KD = 40
_NBD = _ED_T // _KD
_KS = 80
_NBS = 125
_ES_T = _KS * _NBS
_NACC = _N

_BN = 1000
_GRID = _N // _BN


def _mesh():
    return plsc.VectorSubcoreMesh(core_axis_name="c", subcore_axis_name="s")


_SC_PARAMS = pltpu.CompilerParams(use_tc_tiling_on_sc=False)


def _deg_body(dst_hbm, out_hbm, idx_v, ones_v, zb_v, acc_sh):
    c = lax.axis_index("c")
    s = lax.axis_index("s")
    wid = c * _NS + s

    def fill_ones(i, _):
        ones_v[i, :] = jnp.ones((16,), jnp.float32)
        return 0

    lax.fori_loop(0, _KD, fill_ones, 0)

    def fill_zb(i, _):
        zb_v[i, :] = jnp.zeros((16,), jnp.float32)
        return 0

    lax.fori_loop(0, _RPT, fill_zb, 0)

    pltpu.sync_copy(zb_v, acc_sh.at[pl.ds(s * _RPT, _RPT)])
    pltpu.sync_copy(dst_hbm.at[wid], idx_v)
    plsc.subcore_barrier()

    def body(b, _):
        pltpu.sync_copy(ones_v, acc_sh.at[idx_v.at[b]], add=True)
        return 0

    lax.fori_loop(0, _NBD, body, 0)
    plsc.subcore_barrier()

    pltpu.sync_copy(acc_sh.at[pl.ds(s * _RPT, _RPT)], zb_v)
    pltpu.sync_copy(zb_v, out_hbm.at[c, pl.ds(s * _RPT, _RPT)])


def _deg(dst_d):
    call = pl.kernel(
        _deg_body,
        out_type=jax.ShapeDtypeStruct((_NC, _N, 16), jnp.float32),
        mesh=_mesh(),
        scratch_types=[
            pltpu.VMEM((_NBD, _KD), jnp.int32),
            pltpu.VMEM((_KD, 16), jnp.float32),
            pltpu.VMEM((_RPT, 16), jnp.float32),
            pltpu.VMEM_SHARED((_N, 16), jnp.float32),
        ],
        compiler_params=_SC_PARAMS,
    )
    return call(dst_d)


def _scat_body(tbl_hbm, src_hbm, dst_hbm, out_hbm, isrc, idst, rbuf0, rbuf1, rbuf2, acc_sh, semg0, semg1, semg2):
    c = lax.axis_index("c")
    s = lax.axis_index("s")

    def fz(i, _):
        def fz2(j, _):
            rbuf0[i, pl.ds(j * 16, 16)] = jnp.zeros((16,), jnp.float32)
            return 0

        return lax.fori_loop(0, _DH // 16, fz2, 0)

    lax.fori_loop(0, _KS, fz, 0)
    nfull = _RPT // _KS
    tail = _RPT - nfull * _KS
    for k in range(nfull):
        pltpu.sync_copy(rbuf0, acc_sh.at[pl.ds(s * _RPT + k * _KS, _KS)])
    if tail:
        pltpu.sync_copy(
            rbuf0.at[pl.ds(0, tail)],
            acc_sh.at[pl.ds(s * _RPT + nfull * _KS, tail)],
        )

    pltpu.sync_copy(src_hbm.at[s], isrc)
    pltpu.sync_copy(dst_hbm.at[s], idst)

    coff = c * _N

    def off(i, _):
        def off2(j, _):
            isrc[i, pl.ds(j * 16, 16)] = isrc[i, pl.ds(j * 16, 16)] + coff
            return 0

        return lax.fori_loop(0, _KS // 16, off2, 0)

    lax.fori_loop(0, _NBS, off, 0)
    plsc.subcore_barrier()

    pltpu.async_copy(tbl_hbm.at[isrc.at[0]], rbuf0, semg0)
    pltpu.async_copy(tbl_hbm.at[isrc.at[1]], rbuf1, semg1)

    def trip(i, _):
        b = 3 * i
        pltpu.make_async_copy(tbl_hbm.at[isrc.at[b]], rbuf0, semg0).wait()
        pltpu.async_copy(tbl_hbm.at[isrc.at[b + 2]], rbuf2, semg2)
        pltpu.sync_copy(rbuf0, acc_sh.at[idst.at[b]], add=True)
        pltpu.make_async_copy(tbl_hbm.at[isrc.at[b + 1]], rbuf1, semg1).wait()
        pltpu.async_copy(tbl_hbm.at[isrc.at[b + 3]], rbuf0, semg0)
        pltpu.sync_copy(rbuf1, acc_sh.at[idst.at[b + 1]], add=True)
        pltpu.make_async_copy(tbl_hbm.at[isrc.at[b + 2]], rbuf2, semg2).wait()
        pltpu.async_copy(tbl_hbm.at[isrc.at[b + 4]], rbuf1, semg1)
        pltpu.sync_copy(rbuf2, acc_sh.at[idst.at[b + 2]], add=True)
        return 0

    lax.fori_loop(0, (_NBS - 2) // 3, trip, 0)
    pltpu.make_async_copy(tbl_hbm.at[isrc.at[_NBS - 2]], rbuf0, semg0).wait()
    pltpu.sync_copy(rbuf0, acc_sh.at[idst.at[_NBS - 2]], add=True)
    pltpu.make_async_copy(tbl_hbm.at[isrc.at[_NBS - 1]], rbuf1, semg1).wait()
    pltpu.sync_copy(rbuf1, acc_sh.at[idst.at[_NBS - 1]], add=True)
    plsc.subcore_barrier()

    for k in range(nfull):
        rb = rbuf0 if k % 2 == 0 else rbuf1
        pltpu.sync_copy(acc_sh.at[pl.ds(s * _RPT + k * _KS, _KS)], rb)
        pltpu.sync_copy(rb, out_hbm.at[c, pl.ds(s * _RPT + k * _KS, _KS)])
    if tail:
        pltpu.sync_copy(
            acc_sh.at[pl.ds(s * _RPT + nfull * _KS, tail)],
            rbuf1.at[pl.ds(0, tail)],
        )
        pltpu.sync_copy(
            rbuf1.at[pl.ds(0, tail)],
            out_hbm.at[c, pl.ds(s * _RPT + nfull * _KS, tail)],
        )


def _scat(tbl, src_r, dst_r):
    call = pl.kernel(
        _scat_body,
        out_type=jax.ShapeDtypeStruct((_NC, _N, _DH), jnp.float32),
        mesh=_mesh(),
        scratch_types=[
            pltpu.VMEM((_NBS, _KS), jnp.int32),
            pltpu.VMEM((_NBS, _KS), jnp.int32),
            pltpu.VMEM((_KS, _DH), jnp.float32),
            pltpu.VMEM((_KS, _DH), jnp.float32),
            pltpu.VMEM((_KS, _DH), jnp.float32),
            pltpu.VMEM_SHARED((_NACC, _DH), jnp.float32),
            pltpu.SemaphoreType.DMA,
            pltpu.SemaphoreType.DMA,
            pltpu.SemaphoreType.DMA,
        ],
        compiler_params=_SC_PARAMS,
    )
    return call(tbl, src_r, dst_r)


def _dis_of(dp_ref):
    deg = 1.0 + dp_ref[0, :, 0] + dp_ref[1, :, 0]
    return lax.rsqrt(deg)[:, None]


def _k1_body(x_ref, w_ref, dp_ref, o_ref):
    h = jnp.dot(x_ref[...], w_ref[...], preferred_element_type=jnp.float32)
    hp = h * _dis_of(dp_ref)
    o_ref[0] = hp[:, :_DH]
    o_ref[1] = hp[:, _DH:]


def _k1(x, W1, dp):
    return pl.pallas_call(
        _k1_body,
        grid=(_GRID,),
        in_specs=[
            pl.BlockSpec((_BN, _D), lambda i: (i, 0)),
            pl.BlockSpec((_D, _D), lambda i: (0, 0)),
            pl.BlockSpec((_NC, _BN, 16), lambda i: (0, i, 0)),
        ],
        out_specs=pl.BlockSpec((_NC, _BN, _DH), lambda i: (0, i, 0)),
        out_shape=jax.ShapeDtypeStruct((_NC, _N, _DH), jnp.float32),
    )(x, W1, dp)


def _k2_body(s_ref, hp_ref, dp_ref, b_ref, w_ref, o_ref):
    dis = _dis_of(dp_ref)
    ssum = jnp.concatenate([s_ref[0] + hp_ref[0], s_ref[1] + hp_ref[1]], axis=1)
    z = jnp.maximum(ssum * dis + b_ref[...], 0.0)
    h = jnp.dot(z, w_ref[...], preferred_element_type=jnp.float32)
    hp = h * dis
    o_ref[0] = hp[:, :_DH]
    o_ref[1] = hp[:, _DH:]


def _k2(s1, h1p, dp, b1, W2):
    return pl.pallas_call(
        _k2_body,
        grid=(_GRID,),
        in_specs=[
            pl.BlockSpec((_NC, _BN, _DH), lambda i: (0, i, 0)),
            pl.BlockSpec((_NC, _BN, _DH), lambda i: (0, i, 0)),
            pl.BlockSpec((_NC, _BN, 16), lambda i: (0, i, 0)),
            pl.BlockSpec((1, _D), lambda i: (0, 0)),
            pl.BlockSpec((_D, _D), lambda i: (0, 0)),
        ],
        out_specs=pl.BlockSpec((_NC, _BN, _DH), lambda i: (0, i, 0)),
        out_shape=jax.ShapeDtypeStruct((_NC, _N, _DH), jnp.float32),
    )(s1, h1p, dp, b1, W2)


def _k3_body(s_ref, hp_ref, dp_ref, b_ref, o_ref):
    dis = _dis_of(dp_ref)
    ssum = jnp.concatenate([s_ref[0] + hp_ref[0], s_ref[1] + hp_ref[1]], axis=1)
    o_ref[...] = ssum * dis + b_ref[...]


def _k3(s2, h2p, dp, b2):
    return pl.pallas_call(
        _k3_body,
        grid=(_GRID,),
        in_specs=[
            pl.BlockSpec((_NC, _BN, _DH), lambda i: (0, i, 0)),
            pl.BlockSpec((_NC, _BN, _DH), lambda i: (0, i, 0)),
            pl.BlockSpec((_NC, _BN, 16), lambda i: (0, i, 0)),
            pl.BlockSpec((1, _D), lambda i: (0, 0)),
        ],
        out_specs=pl.BlockSpec((_BN, _D), lambda i: (i, 0)),
        out_shape=jax.ShapeDtypeStruct((_N, _D), jnp.float32),
    )(s2, h2p, dp, b2)


def kernel(x, edge_index, W1, b1, W2, b2):
    src = edge_index[0]
    dst = edge_index[1]
    src_r = src.reshape(_NS, _NBS, _KS)
    dst_r = dst.reshape(_NS, _NBS, _KS)
    dst_d = dst.reshape(_NC * _NS, _NBD, _KD)

    degp = _deg(dst_d)
    h1p = _k1(x, W1, degp)
    s1 = _scat(h1p.reshape(_NC * _N, _DH), src_r, dst_r)
    h2p = _k2(s1, h1p, degp, b1.reshape(1, _D), W2)
    s2 = _scat(h2p.reshape(_NC * _N, _DH), src_r, dst_r)
    return _k3(s2, h2p, degp, b2.reshape(1, _D))


# speedup vs baseline: 1.3906x; 1.0019x over previous
import jax
import jax.numpy as jnp
from jax import lax
from jax.experimental import pallas as pl
from jax.experimental.pallas import tpu as pltpu
from jax.experimental.pallas import tpu_sc as plsc

_N = 10000
_E = 160000
_D = 256
_DH = 128
_NC = 2
_NS = 16
_RPT = _N // _NS

_ED_T = _E // (_NC * _NS)
_KD = 40
_NBD = _ED_T // _KD
_KS = 80
_NBS = 125
_ES_T = _KS * _NBS
_NACC = _N

_BN = 1000
_GRID = _N // _BN


def _mesh():
    return plsc.VectorSubcoreMesh(core_axis_name="c", subcore_axis_name="s")


_SC_PARAMS = pltpu.CompilerParams(use_tc_tiling_on_sc=False)


def _deg_body(dst_hbm, out_hbm, idx_v, ones_v, zb_v, acc_sh):
    c = lax.axis_index("c")
    s = lax.axis_index("s")
    wid = c * _NS + s

    def fill_ones(i, _):
        ones_v[i, :] = jnp.ones((16,), jnp.float32)
        return 0

    lax.fori_loop(0, _KD, fill_ones, 0)

    def fill_zb(i, _):
        zb_v[i, :] = jnp.zeros((16,), jnp.float32)
        return 0

    lax.fori_loop(0, _RPT, fill_zb, 0)

    pltpu.sync_copy(zb_v, acc_sh.at[pl.ds(s * _RPT, _RPT)])
    pltpu.sync_copy(dst_hbm.at[wid], idx_v)
    plsc.subcore_barrier()

    def body(b, _):
        pltpu.sync_copy(ones_v, acc_sh.at[idx_v.at[b]], add=True)
        return 0

    lax.fori_loop(0, _NBD, body, 0)
    plsc.subcore_barrier()

    pltpu.sync_copy(
        acc_sh.at[pl.ds(s * _RPT, _RPT)],
        out_hbm.at[c, pl.ds(s * _RPT, _RPT)],
    )


def _deg(dst_d):
    call = pl.kernel(
        _deg_body,
        out_type=jax.ShapeDtypeStruct((_NC, _N, 16), jnp.float32),
        mesh=_mesh(),
        scratch_types=[
            pltpu.VMEM((_NBD, _KD), jnp.int32),
            pltpu.VMEM((_KD, 16), jnp.float32),
            pltpu.VMEM((_RPT, 16), jnp.float32),
            pltpu.VMEM_SHARED((_N, 16), jnp.float32),
        ],
        compiler_params=_SC_PARAMS,
    )
    return call(dst_d)


def _scat_body(tbl_hbm, src_hbm, dst_hbm, out_hbm, isrc, idst, rbuf0, rbuf1, rbuf2, acc_sh, semg0, semg1, semg2):
    c = lax.axis_index("c")
    s = lax.axis_index("s")

    def fz(i, _):
        def fz2(j, _):
            rbuf0[i, pl.ds(j * 16, 16)] = jnp.zeros((16,), jnp.float32)
            return 0

        return lax.fori_loop(0, _DH // 16, fz2, 0)

    lax.fori_loop(0, _KS, fz, 0)
    nfull = _RPT // _KS
    tail = _RPT - nfull * _KS
    for k in range(nfull):
        pltpu.sync_copy(rbuf0, acc_sh.at[pl.ds(s * _RPT + k * _KS, _KS)])
    if tail:
        pltpu.sync_copy(
            rbuf0.at[pl.ds(0, tail)],
            acc_sh.at[pl.ds(s * _RPT + nfull * _KS, tail)],
        )

    pltpu.sync_copy(src_hbm.at[s], isrc)
    pltpu.sync_copy(dst_hbm.at[s], idst)

    coff = c * _N

    def off(i, _):
        def off2(j, _):
            isrc[i, pl.ds(j * 16, 16)] = isrc[i, pl.ds(j * 16, 16)] + coff
            return 0

        return lax.fori_loop(0, _KS // 16, off2, 0)

    lax.fori_loop(0, _NBS, off, 0)
    plsc.subcore_barrier()

    pltpu.async_copy(tbl_hbm.at[isrc.at[0]], rbuf0, semg0)
    pltpu.async_copy(tbl_hbm.at[isrc.at[1]], rbuf1, semg1)

    def trip(i, _):
        b = 3 * i
        pltpu.make_async_copy(tbl_hbm.at[isrc.at[b]], rbuf0, semg0).wait()
        pltpu.async_copy(tbl_hbm.at[isrc.at[b + 2]], rbuf2, semg2)
        pltpu.sync_copy(rbuf0, acc_sh.at[idst.at[b]], add=True)
        pltpu.make_async_copy(tbl_hbm.at[isrc.at[b + 1]], rbuf1, semg1).wait()
        pltpu.async_copy(tbl_hbm.at[isrc.at[b + 3]], rbuf0, semg0)
        pltpu.sync_copy(rbuf1, acc_sh.at[idst.at[b + 1]], add=True)
        pltpu.make_async_copy(tbl_hbm.at[isrc.at[b + 2]], rbuf2, semg2).wait()
        pltpu.async_copy(tbl_hbm.at[isrc.at[b + 4]], rbuf1, semg1)
        pltpu.sync_copy(rbuf2, acc_sh.at[idst.at[b + 2]], add=True)
        return 0

    lax.fori_loop(0, (_NBS - 2) // 3, trip, 0)
    pltpu.make_async_copy(tbl_hbm.at[isrc.at[_NBS - 2]], rbuf0, semg0).wait()
    pltpu.sync_copy(rbuf0, acc_sh.at[idst.at[_NBS - 2]], add=True)
    pltpu.make_async_copy(tbl_hbm.at[isrc.at[_NBS - 1]], rbuf1, semg1).wait()
    pltpu.sync_copy(rbuf1, acc_sh.at[idst.at[_NBS - 1]], add=True)
    plsc.subcore_barrier()

    pltpu.sync_copy(
        acc_sh.at[pl.ds(s * _RPT, _RPT)],
        out_hbm.at[c, pl.ds(s * _RPT, _RPT)],
    )


def _scat(tbl, src_r, dst_r):
    call = pl.kernel(
        _scat_body,
        out_type=jax.ShapeDtypeStruct((_NC, _N, _DH), jnp.float32),
        mesh=_mesh(),
        scratch_types=[
            pltpu.VMEM((_NBS, _KS), jnp.int32),
            pltpu.VMEM((_NBS, _KS), jnp.int32),
            pltpu.VMEM((_KS, _DH), jnp.float32),
            pltpu.VMEM((_KS, _DH), jnp.float32),
            pltpu.VMEM((_KS, _DH), jnp.float32),
            pltpu.VMEM_SHARED((_NACC, _DH), jnp.float32),
            pltpu.SemaphoreType.DMA,
            pltpu.SemaphoreType.DMA,
            pltpu.SemaphoreType.DMA,
        ],
        compiler_params=_SC_PARAMS,
    )
    return call(tbl, src_r, dst_r)


def _dis_of(dp_ref):
    deg = 1.0 + dp_ref[0, :, 0] + dp_ref[1, :, 0]
    return lax.rsqrt(deg)[:, None]


def _k1_body(x_ref, w_ref, dp_ref, o_ref):
    h = jnp.dot(x_ref[...], w_ref[...], preferred_element_type=jnp.float32)
    hp = h * _dis_of(dp_ref)
    o_ref[0] = hp[:, :_DH]
    o_ref[1] = hp[:, _DH:]


def _k1(x, W1, dp):
    return pl.pallas_call(
        _k1_body,
        grid=(_GRID,),
        in_specs=[
            pl.BlockSpec((_BN, _D), lambda i: (i, 0)),
            pl.BlockSpec((_D, _D), lambda i: (0, 0)),
            pl.BlockSpec((_NC, _BN, 16), lambda i: (0, i, 0)),
        ],
        out_specs=pl.BlockSpec((_NC, _BN, _DH), lambda i: (0, i, 0)),
        out_shape=jax.ShapeDtypeStruct((_NC, _N, _DH), jnp.float32),
    )(x, W1, dp)


def _k2_body(s_ref, hp_ref, dp_ref, b_ref, w_ref, o_ref):
    dis = _dis_of(dp_ref)
    ssum = jnp.concatenate([s_ref[0] + hp_ref[0], s_ref[1] + hp_ref[1]], axis=1)
    z = jnp.maximum(ssum * dis + b_ref[...], 0.0)
    h = jnp.dot(z, w_ref[...], preferred_element_type=jnp.float32)
    hp = h * dis
    o_ref[0] = hp[:, :_DH]
    o_ref[1] = hp[:, _DH:]


def _k2(s1, h1p, dp, b1, W2):
    return pl.pallas_call(
        _k2_body,
        grid=(_GRID,),
        in_specs=[
            pl.BlockSpec((_NC, _BN, _DH), lambda i: (0, i, 0)),
            pl.BlockSpec((_NC, _BN, _DH), lambda i: (0, i, 0)),
            pl.BlockSpec((_NC, _BN, 16), lambda i: (0, i, 0)),
            pl.BlockSpec((1, _D), lambda i: (0, 0)),
            pl.BlockSpec((_D, _D), lambda i: (0, 0)),
        ],
        out_specs=pl.BlockSpec((_NC, _BN, _DH), lambda i: (0, i, 0)),
        out_shape=jax.ShapeDtypeStruct((_NC, _N, _DH), jnp.float32),
    )(s1, h1p, dp, b1, W2)


def _k3_body(s_ref, hp_ref, dp_ref, b_ref, o_ref):
    dis = _dis_of(dp_ref)
    ssum = jnp.concatenate([s_ref[0] + hp_ref[0], s_ref[1] + hp_ref[1]], axis=1)
    o_ref[...] = ssum * dis + b_ref[...]


def _k3(s2, h2p, dp, b2):
    return pl.pallas_call(
        _k3_body,
        grid=(_GRID,),
        in_specs=[
            pl.BlockSpec((_NC, _BN, _DH), lambda i: (0, i, 0)),
            pl.BlockSpec((_NC, _BN, _DH), lambda i: (0, i, 0)),
            pl.BlockSpec((_NC, _BN, 16), lambda i: (0, i, 0)),
            pl.BlockSpec((1, _D), lambda i: (0, 0)),
        ],
        out_specs=pl.BlockSpec((_BN, _D), lambda i: (i, 0)),
        out_shape=jax.ShapeDtypeStruct((_N, _D), jnp.float32),
    )(s2, h2p, dp, b2)


def kernel(x, edge_index, W1, b1, W2, b2):
    src = edge_index[0]
    dst = edge_index[1]
    src_r = src.reshape(_NS, _NBS, _KS)
    dst_r = dst.reshape(_NS, _NBS, _KS)
    dst_d = dst.reshape(_NC * _NS, _NBD, _KD)

    degp = _deg(dst_d)
    h1p = _k1(x, W1, degp)
    s1 = _scat(h1p.reshape(_NC * _N, _DH), src_r, dst_r)
    h2p = _k2(s1, h1p, degp, b1.reshape(1, _D), W2)
    s2 = _scat(h2p.reshape(_NC * _N, _DH), src_r, dst_r)
    return _k3(s2, h2p, degp, b2.reshape(1, _D))
